# Initial kernel scaffold; baseline (speedup 1.0000x reference)
import functools

import jax
import jax.numpy as jnp
from jax import lax
from jax.experimental import pallas as pl
from jax.experimental.pallas import tpu as pltpu
from jax.experimental.pallas import tpu_sc as plsc

N = 100000
E = 3200000
D = 128
F = 16
BATCH = 128
NB = E // BATCH
NC, NS = 2, 16
NB_CORE = NB // NC
PER_TILE = NB_CORE // NS
EXTRA = NB_CORE - PER_TILE * NS
CHUNK = 11
NCHUNK = PER_TILE // CHUNK
N_TILE = N // NS

_mesh = plsc.VectorSubcoreMesh(core_axis_name="c", subcore_axis_name="s")


@functools.partial(
    pl.kernel,
    mesh=_mesh,
    out_type=jax.ShapeDtypeStruct((NC, N), jnp.float32),
    scratch_types=[
        pltpu.VMEM((CHUNK, BATCH), jnp.int32),
        pltpu.VMEM((BATCH,), jnp.float32),
        pltpu.VMEM_SHARED((N,), jnp.float32),
        pltpu.SemaphoreType.DMA,
    ],
)
def _sc_degree(dst_hbm, ones_hbm, zeros_hbm, degp_hbm, idxv, onesv, acc, sem):
    c = lax.axis_index("c")
    s = lax.axis_index("s")
    pltpu.sync_copy(ones_hbm, onesv)
    @pl.when(s < 4)
    def _():
        pltpu.sync_copy(zeros_hbm, acc.at[pl.ds(s * 25000, 25000)])
    plsc.subcore_barrier()

    base0 = c * NB_CORE + s * PER_TILE

    def body(i, carry):
        base = base0 + i * CHUNK
        pltpu.sync_copy(dst_hbm.at[pl.ds(base, CHUNK)], idxv)
        for j in range(CHUNK):
            pltpu.sync_copy(onesv, acc.at[idxv.at[j]], add=True)
        return carry

    lax.fori_loop(0, NCHUNK, body, 0)

    @pl.when(s < EXTRA)
    def _():
        base = c * NB_CORE + NS * PER_TILE + s
        pltpu.sync_copy(dst_hbm.at[pl.ds(base, 1)], idxv.at[pl.ds(0, 1)])
        pltpu.sync_copy(onesv, acc.at[idxv.at[0]], add=True)

    plsc.subcore_barrier()
    @pl.when(s < 4)
    def _():
        pltpu.sync_copy(acc.at[pl.ds(s * 25000, 25000)],
                        degp_hbm.at[c, pl.ds(s * 25000, 25000)])


@functools.partial(
    pl.kernel,
    mesh=_mesh,
    out_type=jax.ShapeDtypeStruct((NC, N, F), jnp.float32),
    scratch_types=[
        pltpu.VMEM((CHUNK, BATCH), jnp.int32),
        pltpu.VMEM((CHUNK, BATCH), jnp.int32),
        pltpu.VMEM((CHUNK * BATCH, F), jnp.float32),
        pltpu.VMEM_SHARED((N, F), jnp.float32),
        pltpu.SemaphoreType.DMA,
    ],
)
def _sc_aggregate(src_hbm, dst_hbm, g_hbm, zeros_hbm, aggp_hbm,
                  sidxv, didxv, rows, acc, sem):
    c = lax.axis_index("c")
    s = lax.axis_index("s")
    pltpu.sync_copy(zeros_hbm, acc.at[pl.ds(s * N_TILE, N_TILE)])
    plsc.subcore_barrier()

    base0 = c * NB_CORE + s * PER_TILE

    def body(i, carry):
        base = base0 + i * CHUNK
        pltpu.sync_copy(src_hbm.at[pl.ds(base, CHUNK)], sidxv)
        pltpu.sync_copy(dst_hbm.at[pl.ds(base, CHUNK)], didxv)
        copies = [
            pltpu.async_copy(g_hbm.at[sidxv.at[j]],
                             rows.at[pl.ds(j * BATCH, BATCH)], sem)
            for j in range(CHUNK)
        ]
        for cp in copies:
            cp.wait()
        for j in range(CHUNK):
            pltpu.sync_copy(rows.at[pl.ds(j * BATCH, BATCH)],
                            acc.at[didxv.at[j]], add=True)
        return carry

    lax.fori_loop(0, NCHUNK, body, 0)

    @pl.when(s < EXTRA)
    def _():
        base = c * NB_CORE + NS * PER_TILE + s
        pltpu.sync_copy(src_hbm.at[pl.ds(base, 1)], sidxv.at[pl.ds(0, 1)])
        pltpu.sync_copy(dst_hbm.at[pl.ds(base, 1)], didxv.at[pl.ds(0, 1)])
        pltpu.async_copy(g_hbm.at[sidxv.at[0]],
                         rows.at[pl.ds(0, BATCH)], sem).wait()
        pltpu.sync_copy(rows.at[pl.ds(0, BATCH)], acc.at[didxv.at[0]], add=True)

    plsc.subcore_barrier()
    pltpu.sync_copy(acc.at[pl.ds(s * N_TILE, N_TILE)],
                    aggp_hbm.at[c, pl.ds(s * N_TILE, N_TILE)])


BR = 2000


def _tc1_body(degp_ref, x_ref, w1_ref, g1_ref):
    deg = degp_ref[0, :] + degp_ref[1, :] + 1.0
    dis = lax.rsqrt(deg)
    h = jnp.dot(x_ref[...], w1_ref[...], preferred_element_type=jnp.float32)
    g1_ref[...] = dis[:, None] * h


def _tc2_body(degp_ref, aggp_ref, g1_ref, b1_ref, w2_ref, g2_ref):
    deg = degp_ref[0, :] + degp_ref[1, :] + 1.0
    dis = lax.rsqrt(deg)
    ssum = aggp_ref[0] + aggp_ref[1] + g1_ref[...]
    h = jnp.maximum(dis[:, None] * ssum + b1_ref[...], 0.0)
    g2_ref[...] = dis[:, None] * jnp.dot(h, w2_ref[...],
                                         preferred_element_type=jnp.float32)


def _tc3_body(degp_ref, aggp_ref, g2_ref, b2_ref, out_ref):
    deg = degp_ref[0, :] + degp_ref[1, :] + 1.0
    dis = lax.rsqrt(deg)
    out_ref[...] = dis[:, None] * (aggp_ref[0] + aggp_ref[1] + g2_ref[...]) \
        + b2_ref[...]


def kernel(x, edge_index, W1, b1, W2, b2):
    src = edge_index[0].reshape(NB, BATCH)
    dst = edge_index[1].reshape(NB, BATCH)
    ones_row = jnp.ones((BATCH,), jnp.float32)
    zeros_deg = jnp.zeros((25000,), jnp.float32)
    zeros_agg = jnp.zeros((N_TILE, F), jnp.float32)
    b1r = b1.reshape(1, F)
    b2r = b2.reshape(1, F)

    degp = _sc_degree(dst, ones_row, zeros_deg)

    nblk = N // BR
    g1 = pl.pallas_call(
        _tc1_body,
        grid=(nblk,),
        in_specs=[
            pl.BlockSpec((2, BR), lambda i: (0, i)),
            pl.BlockSpec((BR, D), lambda i: (i, 0)),
            pl.BlockSpec((D, F), lambda i: (0, 0)),
        ],
        out_specs=pl.BlockSpec((BR, F), lambda i: (i, 0)),
        out_shape=jax.ShapeDtypeStruct((N, F), jnp.float32),
    )(degp, x, W1)

    aggp1 = _sc_aggregate(src, dst, g1, zeros_agg)

    g2 = pl.pallas_call(
        _tc2_body,
        grid=(nblk,),
        in_specs=[
            pl.BlockSpec((2, BR), lambda i: (0, i)),
            pl.BlockSpec((2, BR, F), lambda i: (0, i, 0)),
            pl.BlockSpec((BR, F), lambda i: (i, 0)),
            pl.BlockSpec((1, F), lambda i: (0, 0)),
            pl.BlockSpec((F, F), lambda i: (0, 0)),
        ],
        out_specs=pl.BlockSpec((BR, F), lambda i: (i, 0)),
        out_shape=jax.ShapeDtypeStruct((N, F), jnp.float32),
    )(degp, aggp1, g1, b1r, W2)

    aggp2 = _sc_aggregate(src, dst, g2, zeros_agg)

    out = pl.pallas_call(
        _tc3_body,
        grid=(nblk,),
        in_specs=[
            pl.BlockSpec((2, BR), lambda i: (0, i)),
            pl.BlockSpec((2, BR, F), lambda i: (0, i, 0)),
            pl.BlockSpec((BR, F), lambda i: (i, 0)),
            pl.BlockSpec((1, F), lambda i: (0, 0)),
        ],
        out_specs=pl.BlockSpec((BR, F), lambda i: (i, 0)),
        out_shape=jax.ShapeDtypeStruct((N, F), jnp.float32),
    )(degp, aggp2, g2, b2r)

    return out


# speedup vs baseline: 6.2952x; 6.2952x over previous
import functools

import jax
import jax.numpy as jnp
from jax import lax
from jax.experimental import pallas as pl
from jax.experimental.pallas import tpu as pltpu
from jax.experimental.pallas import tpu_sc as plsc

N = 100000
NP = 100352
E = 3200000
D = 128
F = 16
NH = NP // 2
DUMP = 128
ACCR = NH + DUMP
NS = 16
CE = 2000
NCHE = E // CE
PER_T = NCHE // NS
SUB = 5
NGRP = CE // 16
NSUB = NGRP // SUB
MSPAN = NH // NS
MS = MSPAN // 4
ZR = 1048

_mesh = plsc.VectorSubcoreMesh(core_axis_name="c", subcore_axis_name="s",
                               num_cores=1)


def _newton_rsqrt(d):
    m = jnp.full_like(d, 1.0)
    em = jnp.full_like(d, 1.0)
    for j in range(1, 23):
        crossed = d >= jnp.float32(float(2 ** j))
        m = m * jnp.where(crossed, jnp.float32(0.7071067811865476),
                          jnp.float32(1.0))
        em = em * jnp.where(crossed, jnp.float32(0.5), jnp.float32(1.0))
    dn = d * em
    y = m * (1.4274 - 0.3015 * dn)
    for _ in range(3):
        y = y * (1.5 - 0.5 * d * y * y)
    return y


@functools.partial(
    pl.kernel,
    mesh=_mesh,
    compiler_params=pltpu.CompilerParams(use_tc_tiling_on_sc=False),
    out_type=[jax.ShapeDtypeStruct((NP, F), jnp.float32)] * 4,
    scratch_types=[
        pltpu.VMEM((CE,), jnp.int32),
        pltpu.VMEM((CE,), jnp.int32),
        pltpu.VMEM((SUB * 16, F), jnp.float32),
        pltpu.VMEM((MS, F), jnp.float32),
        pltpu.VMEM((MS, F), jnp.float32),
        pltpu.VMEM((MS, F), jnp.float32),
        pltpu.VMEM((MS // 8, 8 * F), jnp.float32),
        pltpu.VMEM((16, F), jnp.float32),
        pltpu.VMEM((1, F), jnp.float32),
        pltpu.VMEM((ZR, F), jnp.float32),
        pltpu.VMEM_SHARED((ACCR, F), jnp.float32),
        pltpu.SemaphoreType.DMA,
    ],
)
def _sc_mega(src_hbm, dst_hbm, h1_hbm, ones_hbm, z_hbm, b1_hbm,
             disb_hbm, t1_hbm, r_hbm, a2_hbm,
             sv, dv, rows, bufa, bufb, bufc, bufh, onev, b1v, zbuf, acc, sem):
    s = lax.axis_index("s")

    def remap(d16, base):
        inr = jnp.logical_and(d16 >= base, d16 < base + NH)
        return jnp.where(inr, d16 - base, NH + (d16 & (DUMP - 1)))

    def deg_chunk(base, chunk):
        off = chunk * CE
        pltpu.sync_copy(dst_hbm.at[pl.ds(off, CE)], dv)

        def body(g, carry):
            d16 = remap(dv[pl.ds(g * 16, 16)], base)
            pltpu.sync_copy(onev, acc.at[d16], add=True)
            return carry

        lax.fori_loop(0, NGRP, body, 0)

    def agg_chunk(tab_hbm, base, chunk):
        off = chunk * CE
        pltpu.sync_copy(src_hbm.at[pl.ds(off, CE)], sv)
        pltpu.sync_copy(dst_hbm.at[pl.ds(off, CE)], dv)

        def sub_body(t, carry):
            g0 = t * SUB
            copies = []
            for k in range(SUB):
                s16 = sv[pl.ds((g0 + k) * 16, 16)]
                copies.append(pltpu.async_copy(
                    tab_hbm.at[s16], rows.at[pl.ds(k * 16, 16)], sem))
            for cp in copies:
                cp.wait()
            for k in range(SUB):
                d16 = remap(dv[pl.ds((g0 + k) * 16, 16)], base)
                pltpu.sync_copy(rows.at[pl.ds(k * 16, 16)],
                                acc.at[d16], add=True)
            return carry

        lax.fori_loop(0, NSUB, sub_body, 0)

    def edge_pass(per_chunk):
        def body(i, carry):
            per_chunk(s * PER_T + i)
            return carry
        lax.fori_loop(0, PER_T, body, 0)

    pltpu.sync_copy(ones_hbm, onev)
    pltpu.sync_copy(b1_hbm, b1v)
    pltpu.sync_copy(z_hbm, zbuf)

    def zero_phase():
        for piece in range(3):
            pltpu.sync_copy(zbuf, acc.at[pl.ds(s * 3 * ZR + piece * ZR, ZR)])

    for p in range(2):
        base = p * NH
        zero_phase()
        plsc.subcore_barrier()
        edge_pass(functools.partial(deg_chunk, base))
        plsc.subcore_barrier()
        for q in range(4):
            loc = s * MSPAN + q * MS
            glob = base + loc
            pltpu.sync_copy(acc.at[pl.ds(loc, MS)], bufa)
            pltpu.sync_copy(h1_hbm.at[pl.ds(glob // 8, MS // 8)], bufh)

            def cbody(i, carry):
                dis = _newton_rsqrt(bufa[i, :] + 1.0)
                bufa[i, :] = dis
                hv = bufh[i >> 3, pl.ds((i & 7) * F, F)]
                bufc[i, :] = hv * dis
                return carry

            lax.fori_loop(0, MS, cbody, 0)
            pltpu.sync_copy(bufa, disb_hbm.at[pl.ds(glob, MS)])
            pltpu.sync_copy(bufc, t1_hbm.at[pl.ds(glob, MS)])
        plsc.subcore_barrier()

    for p in range(2):
        base = p * NH
        zero_phase()
        plsc.subcore_barrier()
        edge_pass(functools.partial(agg_chunk, t1_hbm, base))
        plsc.subcore_barrier()
        for q in range(4):
            loc = s * MSPAN + q * MS
            glob = base + loc
            pltpu.sync_copy(acc.at[pl.ds(loc, MS)], bufa)
            pltpu.sync_copy(t1_hbm.at[pl.ds(glob, MS)], bufb)
            pltpu.sync_copy(disb_hbm.at[pl.ds(glob, MS)], bufc)

            def ebody(i, carry):
                dis = bufc[i, :]
                r = dis * jnp.maximum(
                    dis * (bufa[i, :] + bufb[i, :]) + b1v[0, :], 0.0)
                bufa[i, :] = r
                return carry

            lax.fori_loop(0, MS, ebody, 0)
            pltpu.sync_copy(bufa, r_hbm.at[pl.ds(glob, MS)])
        plsc.subcore_barrier()

    for p in range(2):
        base = p * NH
        zero_phase()
        plsc.subcore_barrier()
        edge_pass(functools.partial(agg_chunk, r_hbm, base))
        plsc.subcore_barrier()
        for q in range(4):
            loc = s * MSPAN + q * MS
            glob = base + loc
            pltpu.sync_copy(acc.at[pl.ds(loc, MS)], bufa)
            pltpu.sync_copy(bufa, a2_hbm.at[pl.ds(glob, MS)])
        plsc.subcore_barrier()


BR = 2048


def _tc1_body(x8_ref, w1b_ref, h_ref):
    h_ref[...] = jnp.dot(x8_ref[...], w1b_ref[...],
                         preferred_element_type=jnp.float32)


def _tc2_body(disb_ref, r_ref, a2_ref, w2b_ref, b2t_ref, out_ref):
    ssum = a2_ref[...] + r_ref[...]
    out_ref[...] = disb_ref[...] * jnp.dot(
        ssum, w2b_ref[...], preferred_element_type=jnp.float32) \
        + b2t_ref[...]


def kernel(x, edge_index, W1, b1, W2, b2):
    src1 = edge_index[0]
    dst1 = edge_index[1]
    ones16 = jnp.ones((16, F), jnp.float32)
    zeros = jnp.zeros((ZR, F), jnp.float32)
    b1r = b1.reshape(1, F)

    nblk = NP // BR
    x8 = x.reshape(N // 8, 8 * D)
    w1big = jnp.kron(jnp.eye(8, dtype=jnp.float32), W1)
    h1p = pl.pallas_call(
        _tc1_body,
        grid=(nblk,),
        in_specs=[
            pl.BlockSpec((BR // 8, 8 * D), lambda i: (i, 0)),
            pl.BlockSpec((8 * D, 8 * F), lambda i: (0, 0)),
        ],
        out_specs=pl.BlockSpec((BR // 8, 8 * F), lambda i: (i, 0)),
        out_shape=jax.ShapeDtypeStruct((NP // 8, 8 * F), jnp.float32),
    )(x8, w1big)

    disb, t1, r, a2 = _sc_mega(src1, dst1, h1p, ones16, zeros, b1r)

    w2big = jnp.kron(jnp.eye(8, dtype=jnp.float32), W2)
    b2t = jnp.tile(b2, 8).reshape(1, 8 * F)
    disp = disb.reshape(NP // 8, 8 * F)
    rp = r.reshape(NP // 8, 8 * F)
    a2p = a2.reshape(NP // 8, 8 * F)
    out = pl.pallas_call(
        _tc2_body,
        grid=(nblk,),
        in_specs=[
            pl.BlockSpec((BR // 8, 8 * F), lambda i: (i, 0)),
            pl.BlockSpec((BR // 8, 8 * F), lambda i: (i, 0)),
            pl.BlockSpec((BR // 8, 8 * F), lambda i: (i, 0)),
            pl.BlockSpec((8 * F, 8 * F), lambda i: (0, 0)),
            pl.BlockSpec((1, 8 * F), lambda i: (0, 0)),
        ],
        out_specs=pl.BlockSpec((BR // 8, 8 * F), lambda i: (i, 0)),
        out_shape=jax.ShapeDtypeStruct((NP // 8, 8 * F), jnp.float32),
    )(disp, rp, a2p, w2big, b2t)

    return out.reshape(NP, F)[:N]


# speedup vs baseline: 7.3033x; 1.1601x over previous
import functools

import jax
import jax.numpy as jnp
from jax import lax
from jax.experimental import pallas as pl
from jax.experimental.pallas import tpu as pltpu
from jax.experimental.pallas import tpu_sc as plsc

N = 100000
NP = 100352
E = 3200000
D = 128
F = 16
NH = NP // 2
DUMP = 128
ACCR = NH + DUMP
NS = 16
BATCH = 128
NB = E // BATCH
CS = 4
NCHE = NB // CS
PER_T = NCHE // NS
EXTRA = NCHE - PER_T * NS
MSPAN = NH // NS
MS = MSPAN // 4
ZR = 1048

_mesh = plsc.VectorSubcoreMesh(core_axis_name="c", subcore_axis_name="s",
                               num_cores=1)


def _newton_rsqrt(d):
    m = jnp.full_like(d, 1.0)
    em = jnp.full_like(d, 1.0)
    for j in range(1, 23):
        crossed = d >= jnp.float32(float(2 ** j))
        m = m * jnp.where(crossed, jnp.float32(0.7071067811865476),
                          jnp.float32(1.0))
        em = em * jnp.where(crossed, jnp.float32(0.5), jnp.float32(1.0))
    dn = d * em
    y = m * (1.4274 - 0.3015 * dn)
    for _ in range(3):
        y = y * (1.5 - 0.5 * d * y * y)
    return y


@functools.partial(
    pl.kernel,
    mesh=_mesh,
    compiler_params=pltpu.CompilerParams(use_tc_tiling_on_sc=False),
    out_type=[jax.ShapeDtypeStruct((NP, F), jnp.float32)] * 4,
    scratch_types=[
        pltpu.VMEM((CS, BATCH), jnp.int32),
        pltpu.VMEM((CS, BATCH), jnp.int32),
        pltpu.VMEM((CS, BATCH), jnp.int32),
        pltpu.VMEM((CS * BATCH, F), jnp.float32),
        pltpu.VMEM((MS, F), jnp.float32),
        pltpu.VMEM((MS, F), jnp.float32),
        pltpu.VMEM((MS, F), jnp.float32),
        pltpu.VMEM((MS // 8, 8 * F), jnp.float32),
        pltpu.VMEM((BATCH, F), jnp.float32),
        pltpu.VMEM((1, F), jnp.float32),
        pltpu.VMEM((ZR, F), jnp.float32),
        pltpu.VMEM_SHARED((ACCR, F), jnp.float32),
        pltpu.SemaphoreType.DMA,
    ],
)
def _sc_mega(src_hbm, dst_hbm, h1_hbm, ones_hbm, z_hbm, b1_hbm,
             disb_hbm, t1_hbm, r_hbm, a2_hbm,
             sidx2, draw2, didx2, rows, bufa, bufb, bufc, bufh, onev, b1v,
             zbuf, acc, sem):
    s = lax.axis_index("s")

    def remap(d16, base):
        inr = jnp.logical_and(d16 >= base, d16 < base + NH)
        return jnp.where(inr, d16 - base, NH + (d16 & (DUMP - 1)))

    def load_remap_dst(base, chunk):
        for j in range(CS):
            pltpu.sync_copy(dst_hbm.at[pl.ds((chunk * CS + j) * BATCH, BATCH)],
                            draw2.at[j])

        def rbody(g, carry):
            j = g >> 3
            u = g & 7
            d16 = remap(draw2[j, pl.ds(u * 16, 16)], base)
            didx2[j, pl.ds(u * 16, 16)] = d16
            return carry

        lax.fori_loop(0, CS * 8, rbody, 0)

    def deg_chunk(base, chunk):
        load_remap_dst(base, chunk)
        for j in range(CS):
            pltpu.sync_copy(onev, acc.at[didx2.at[j]], add=True)

    def agg_chunk(tab_hbm, base, chunk):
        for j in range(CS):
            pltpu.sync_copy(src_hbm.at[pl.ds((chunk * CS + j) * BATCH, BATCH)],
                            sidx2.at[j])
        load_remap_dst(base, chunk)
        copies = [
            pltpu.async_copy(tab_hbm.at[sidx2.at[j]],
                             rows.at[pl.ds(j * BATCH, BATCH)], sem)
            for j in range(CS)
        ]
        for cp in copies:
            cp.wait()
        for j in range(CS):
            pltpu.sync_copy(rows.at[pl.ds(j * BATCH, BATCH)],
                            acc.at[didx2.at[j]], add=True)

    def edge_pass(per_chunk):
        def body(i, carry):
            per_chunk(s * PER_T + i)
            return carry
        lax.fori_loop(0, PER_T, body, 0)

        @pl.when(s < EXTRA)
        def _():
            per_chunk(NS * PER_T + s)

    pltpu.sync_copy(ones_hbm, onev)
    pltpu.sync_copy(b1_hbm, b1v)
    pltpu.sync_copy(z_hbm, zbuf)

    def zero_phase():
        for piece in range(3):
            pltpu.sync_copy(zbuf, acc.at[pl.ds(s * 3 * ZR + piece * ZR, ZR)])

    for p in range(2):
        base = p * NH
        zero_phase()
        plsc.subcore_barrier()
        edge_pass(functools.partial(deg_chunk, base))
        plsc.subcore_barrier()
        for q in range(4):
            loc = s * MSPAN + q * MS
            glob = base + loc
            pltpu.sync_copy(acc.at[pl.ds(loc, MS)], bufa)
            pltpu.sync_copy(h1_hbm.at[pl.ds(glob // 8, MS // 8)], bufh)

            def cbody(i, carry):
                dis = _newton_rsqrt(bufa[i, :] + 1.0)
                bufa[i, :] = dis
                hv = bufh[i >> 3, pl.ds((i & 7) * F, F)]
                bufc[i, :] = hv * dis
                return carry

            lax.fori_loop(0, MS, cbody, 0)
            pltpu.sync_copy(bufa, disb_hbm.at[pl.ds(glob, MS)])
            pltpu.sync_copy(bufc, t1_hbm.at[pl.ds(glob, MS)])
        plsc.subcore_barrier()

    for p in range(2):
        base = p * NH
        zero_phase()
        plsc.subcore_barrier()
        edge_pass(functools.partial(agg_chunk, t1_hbm, base))
        plsc.subcore_barrier()
        for q in range(4):
            loc = s * MSPAN + q * MS
            glob = base + loc
            pltpu.sync_copy(acc.at[pl.ds(loc, MS)], bufa)
            pltpu.sync_copy(t1_hbm.at[pl.ds(glob, MS)], bufb)
            pltpu.sync_copy(disb_hbm.at[pl.ds(glob, MS)], bufc)

            def ebody(i, carry):
                dis = bufc[i, :]
                r = dis * jnp.maximum(
                    dis * (bufa[i, :] + bufb[i, :]) + b1v[0, :], 0.0)
                bufa[i, :] = r
                return carry

            lax.fori_loop(0, MS, ebody, 0)
            pltpu.sync_copy(bufa, r_hbm.at[pl.ds(glob, MS)])
        plsc.subcore_barrier()

    for p in range(2):
        base = p * NH
        zero_phase()
        plsc.subcore_barrier()
        edge_pass(functools.partial(agg_chunk, r_hbm, base))
        plsc.subcore_barrier()
        for q in range(4):
            loc = s * MSPAN + q * MS
            glob = base + loc
            pltpu.sync_copy(acc.at[pl.ds(loc, MS)], bufa)
            pltpu.sync_copy(bufa, a2_hbm.at[pl.ds(glob, MS)])
        plsc.subcore_barrier()


BR = 2048


def _tc1_body(x8_ref, w1b_ref, h_ref):
    h_ref[...] = jnp.dot(x8_ref[...], w1b_ref[...],
                         preferred_element_type=jnp.float32)


def _tc2_body(disb_ref, r_ref, a2_ref, w2b_ref, b2t_ref, out_ref):
    ssum = a2_ref[...] + r_ref[...]
    out_ref[...] = disb_ref[...] * jnp.dot(
        ssum, w2b_ref[...], preferred_element_type=jnp.float32) \
        + b2t_ref[...]


def kernel(x, edge_index, W1, b1, W2, b2):
    src1 = edge_index[0]
    dst1 = edge_index[1]
    ones16 = jnp.ones((BATCH, F), jnp.float32)
    zeros = jnp.zeros((ZR, F), jnp.float32)
    b1r = b1.reshape(1, F)

    nblk = NP // BR
    x8 = x.reshape(N // 8, 8 * D)
    w1big = jnp.kron(jnp.eye(8, dtype=jnp.float32), W1)
    h1p = pl.pallas_call(
        _tc1_body,
        grid=(nblk,),
        in_specs=[
            pl.BlockSpec((BR // 8, 8 * D), lambda i: (i, 0)),
            pl.BlockSpec((8 * D, 8 * F), lambda i: (0, 0)),
        ],
        out_specs=pl.BlockSpec((BR // 8, 8 * F), lambda i: (i, 0)),
        out_shape=jax.ShapeDtypeStruct((NP // 8, 8 * F), jnp.float32),
    )(x8, w1big)

    disb, t1, r, a2 = _sc_mega(src1, dst1, h1p, ones16, zeros, b1r)

    w2big = jnp.kron(jnp.eye(8, dtype=jnp.float32), W2)
    b2t = jnp.tile(b2, 8).reshape(1, 8 * F)
    disp = disb.reshape(NP // 8, 8 * F)
    rp = r.reshape(NP // 8, 8 * F)
    a2p = a2.reshape(NP // 8, 8 * F)
    out = pl.pallas_call(
        _tc2_body,
        grid=(nblk,),
        in_specs=[
            pl.BlockSpec((BR // 8, 8 * F), lambda i: (i, 0)),
            pl.BlockSpec((BR // 8, 8 * F), lambda i: (i, 0)),
            pl.BlockSpec((BR // 8, 8 * F), lambda i: (i, 0)),
            pl.BlockSpec((8 * F, 8 * F), lambda i: (0, 0)),
            pl.BlockSpec((1, 8 * F), lambda i: (0, 0)),
        ],
        out_specs=pl.BlockSpec((BR // 8, 8 * F), lambda i: (i, 0)),
        out_shape=jax.ShapeDtypeStruct((NP // 8, 8 * F), jnp.float32),
    )(disp, rp, a2p, w2big, b2t)

    return out.reshape(NP, F)[:N]


# speedup vs baseline: 9.9449x; 1.3617x over previous
import functools

import jax
import jax.numpy as jnp
from jax import lax
from jax.experimental import pallas as pl
from jax.experimental.pallas import tpu as pltpu
from jax.experimental.pallas import tpu_sc as plsc

N = 100000
NP = 100352
E = 3200000
D = 128
F = 16
NH = NP // 2
DUMP = 128
ACCR = NH + DUMP
NS = 16
BATCH = 128
NB = E // BATCH
CS = 4
NCHE = NB // CS
PER_T = NCHE // NS
EXTRA = NCHE - PER_T * NS
MSPAN = NH // NS
MS = MSPAN // 4
ZR = 1048

_mesh = plsc.VectorSubcoreMesh(core_axis_name="c", subcore_axis_name="s",
                               num_cores=1)


def _newton_rsqrt(d):
    m = jnp.full_like(d, 1.0)
    em = jnp.full_like(d, 1.0)
    for j in range(1, 23):
        crossed = d >= jnp.float32(float(2 ** j))
        m = m * jnp.where(crossed, jnp.float32(0.7071067811865476),
                          jnp.float32(1.0))
        em = em * jnp.where(crossed, jnp.float32(0.5), jnp.float32(1.0))
    dn = d * em
    y = m * (1.4274 - 0.3015 * dn)
    for _ in range(3):
        y = y * (1.5 - 0.5 * d * y * y)
    return y


@functools.partial(
    pl.kernel,
    mesh=_mesh,
    compiler_params=pltpu.CompilerParams(use_tc_tiling_on_sc=False),
    out_type=[jax.ShapeDtypeStruct((NP, F), jnp.float32)] * 4,
    scratch_types=[
        pltpu.VMEM((2 * CS, BATCH), jnp.int32),
        pltpu.VMEM((2 * CS, BATCH), jnp.int32),
        pltpu.VMEM((2 * CS, BATCH), jnp.int32),
        pltpu.VMEM((2 * CS * BATCH, F), jnp.float32),
        pltpu.VMEM((MS, F), jnp.float32),
        pltpu.VMEM((MS, F), jnp.float32),
        pltpu.VMEM((MS, F), jnp.float32),
        pltpu.VMEM((MS // 8, 8 * F), jnp.float32),
        pltpu.VMEM((BATCH, F), jnp.float32),
        pltpu.VMEM((1, F), jnp.float32),
        pltpu.VMEM_SHARED((ACCR, F), jnp.float32),
        pltpu.SemaphoreType.DMA,
        pltpu.SemaphoreType.DMA,
        pltpu.SemaphoreType.DMA,
    ],
)
def _sc_mega(src_hbm, dst_hbm, h1_hbm, ones_hbm, b1_hbm,
             disb_hbm, t1_hbm, r_hbm, a2_hbm,
             sidx2, draw2, didx2, rows, bufa, bufb, bufc, bufh, onev, b1v,
             acc, sem, ssem0, ssem1):
    s = lax.axis_index("s")

    def remap(d16, base):
        inr = jnp.logical_and(d16 >= base, d16 < base + NH)
        return jnp.where(inr, d16 - base, NH + (d16 & (DUMP - 1)))

    def load_remap_dst(base, chunk, h):
        jo = h * CS
        for j in range(CS):
            pltpu.sync_copy(dst_hbm.at[pl.ds((chunk * CS + j) * BATCH, BATCH)],
                            draw2.at[jo + j])

        def rbody(g, carry):
            j = jo + (g >> 3)
            u = g & 7
            d16 = remap(draw2[j, pl.ds(u * 16, 16)], base)
            didx2[j, pl.ds(u * 16, 16)] = d16
            return carry

        lax.fori_loop(0, CS * 8, rbody, 0)

    def drain_scatters(h):
        ssem = ssem0 if h == 0 else ssem1
        ro = h * CS * BATCH
        for j in range(CS):
            pltpu.make_async_copy(
                t1_hbm.at[pl.ds(0, BATCH)],
                rows.at[pl.ds(ro + j * BATCH, BATCH)], ssem).wait()

    def deg_half(base, chunk, h):
        ssem = ssem0 if h == 0 else ssem1
        jo = h * CS
        load_remap_dst(base, chunk, h)
        for j in range(CS):
            pltpu.async_copy(onev, acc.at[didx2.at[jo + j]], ssem, add=True)

    def agg_half(tab_hbm, base, chunk, h):
        ssem = ssem0 if h == 0 else ssem1
        jo = h * CS
        ro = h * CS * BATCH
        for j in range(CS):
            pltpu.sync_copy(src_hbm.at[pl.ds((chunk * CS + j) * BATCH, BATCH)],
                            sidx2.at[jo + j])
        copies = [
            pltpu.async_copy(tab_hbm.at[sidx2.at[jo + j]],
                             rows.at[pl.ds(ro + j * BATCH, BATCH)], sem)
            for j in range(CS)
        ]
        load_remap_dst(base, chunk, h)
        for cp in copies:
            cp.wait()
        for j in range(CS):
            pltpu.async_copy(rows.at[pl.ds(ro + j * BATCH, BATCH)],
                             acc.at[didx2.at[jo + j]], ssem, add=True)

    def edge_pass(half_fn):
        c0 = s * PER_T
        half_fn(c0, 0)
        half_fn(c0 + 1, 1)

        def body(i, carry):
            drain_scatters(0)
            half_fn(c0 + 2 * i, 0)
            drain_scatters(1)
            half_fn(c0 + 2 * i + 1, 1)
            return carry

        lax.fori_loop(1, PER_T // 2, body, 0)
        drain_scatters(0)
        drain_scatters(1)

        @pl.when(s < EXTRA)
        def _():
            half_fn(NS * PER_T + s, 0)
            drain_scatters(0)

    pltpu.sync_copy(ones_hbm, onev)
    pltpu.sync_copy(b1_hbm, b1v)

    def zero_phase():
        def zb(i, carry):
            bufa[i, :] = jnp.zeros((F,), jnp.float32)
            return carry
        lax.fori_loop(0, MS, zb, 0)
        for piece in range(4):
            pltpu.sync_copy(bufa.at[pl.ds(0, MS)],
                            acc.at[pl.ds(s * 3144 + piece * MS, MS)])
        pltpu.sync_copy(bufa.at[pl.ds(0, 8)],
                        acc.at[pl.ds(s * 3144 + 4 * MS, 8)])

    for p in range(2):
        base = p * NH
        zero_phase()
        plsc.subcore_barrier()
        edge_pass(functools.partial(deg_half, base))
        plsc.subcore_barrier()
        for q in range(4):
            loc = s * MSPAN + q * MS
            glob = base + loc
            pltpu.sync_copy(acc.at[pl.ds(loc, MS)], bufa)
            pltpu.sync_copy(h1_hbm.at[pl.ds(glob // 8, MS // 8)], bufh)

            def cbody(i, carry):
                dis = _newton_rsqrt(bufa[i, :] + 1.0)
                bufa[i, :] = dis
                hv = bufh[i >> 3, pl.ds((i & 7) * F, F)]
                bufc[i, :] = hv * dis
                return carry

            lax.fori_loop(0, MS, cbody, 0)
            pltpu.sync_copy(bufa, disb_hbm.at[pl.ds(glob, MS)])
            pltpu.sync_copy(bufc, t1_hbm.at[pl.ds(glob, MS)])
        plsc.subcore_barrier()

    for p in range(2):
        base = p * NH
        zero_phase()
        plsc.subcore_barrier()
        edge_pass(functools.partial(agg_half, t1_hbm, base))
        plsc.subcore_barrier()
        for q in range(4):
            loc = s * MSPAN + q * MS
            glob = base + loc
            pltpu.sync_copy(acc.at[pl.ds(loc, MS)], bufa)
            pltpu.sync_copy(t1_hbm.at[pl.ds(glob, MS)], bufb)
            pltpu.sync_copy(disb_hbm.at[pl.ds(glob, MS)], bufc)

            def ebody(i, carry):
                dis = bufc[i, :]
                r = dis * jnp.maximum(
                    dis * (bufa[i, :] + bufb[i, :]) + b1v[0, :], 0.0)
                bufa[i, :] = r
                return carry

            lax.fori_loop(0, MS, ebody, 0)
            pltpu.sync_copy(bufa, r_hbm.at[pl.ds(glob, MS)])
        plsc.subcore_barrier()

    for p in range(2):
        base = p * NH
        zero_phase()
        plsc.subcore_barrier()
        edge_pass(functools.partial(agg_half, r_hbm, base))
        plsc.subcore_barrier()
        for q in range(4):
            loc = s * MSPAN + q * MS
            glob = base + loc
            pltpu.sync_copy(acc.at[pl.ds(loc, MS)], bufa)
            pltpu.sync_copy(bufa, a2_hbm.at[pl.ds(glob, MS)])
        plsc.subcore_barrier()


BR = 2048


def _tc1_body(x8_ref, w1b_ref, h_ref):
    h_ref[...] = jnp.dot(x8_ref[...], w1b_ref[...],
                         preferred_element_type=jnp.float32)


def _tc2_body(disb_ref, r_ref, a2_ref, w2b_ref, b2t_ref, out_ref):
    ssum = a2_ref[...] + r_ref[...]
    out_ref[...] = disb_ref[...] * jnp.dot(
        ssum, w2b_ref[...], preferred_element_type=jnp.float32) \
        + b2t_ref[...]


def kernel(x, edge_index, W1, b1, W2, b2):
    src1 = edge_index[0]
    dst1 = edge_index[1]
    ones16 = jnp.ones((BATCH, F), jnp.float32)
    b1r = b1.reshape(1, F)

    nblk = NP // BR
    x8 = x.reshape(N // 8, 8 * D)
    w1big = jnp.kron(jnp.eye(8, dtype=jnp.float32), W1)
    h1p = pl.pallas_call(
        _tc1_body,
        grid=(nblk,),
        in_specs=[
            pl.BlockSpec((BR // 8, 8 * D), lambda i: (i, 0)),
            pl.BlockSpec((8 * D, 8 * F), lambda i: (0, 0)),
        ],
        out_specs=pl.BlockSpec((BR // 8, 8 * F), lambda i: (i, 0)),
        out_shape=jax.ShapeDtypeStruct((NP // 8, 8 * F), jnp.float32),
    )(x8, w1big)

    disb, t1, r, a2 = _sc_mega(src1, dst1, h1p, ones16, b1r)

    w2big = jnp.kron(jnp.eye(8, dtype=jnp.float32), W2)
    b2t = jnp.tile(b2, 8).reshape(1, 8 * F)
    disp = disb.reshape(NP // 8, 8 * F)
    rp = r.reshape(NP // 8, 8 * F)
    a2p = a2.reshape(NP // 8, 8 * F)
    out = pl.pallas_call(
        _tc2_body,
        grid=(nblk,),
        in_specs=[
            pl.BlockSpec((BR // 8, 8 * F), lambda i: (i, 0)),
            pl.BlockSpec((BR // 8, 8 * F), lambda i: (i, 0)),
            pl.BlockSpec((BR // 8, 8 * F), lambda i: (i, 0)),
            pl.BlockSpec((8 * F, 8 * F), lambda i: (0, 0)),
            pl.BlockSpec((1, 8 * F), lambda i: (0, 0)),
        ],
        out_specs=pl.BlockSpec((BR // 8, 8 * F), lambda i: (i, 0)),
        out_shape=jax.ShapeDtypeStruct((NP // 8, 8 * F), jnp.float32),
    )(disp, rp, a2p, w2big, b2t)

    return out.reshape(NP, F)[:N]


# speedup vs baseline: 20.6021x; 2.0716x over previous
import functools

import jax
import jax.numpy as jnp
from jax import lax
from jax.experimental import pallas as pl
from jax.experimental.pallas import tpu as pltpu
from jax.experimental.pallas import tpu_sc as plsc

N = 100000
NP = 100352
E = 3200000
D = 128
F = 16
NH = NP // 2
DUMP = 128
ACCR = NH + DUMP
NS = 16
BATCH = 128
NB = E // BATCH
CS = 4
NCHE = NB // CS
PER_T = NCHE // NS
EXTRA = NCHE - PER_T * NS
MSPAN = NH // NS
MS = MSPAN // 4
ZR = 1048

_mesh = plsc.VectorSubcoreMesh(core_axis_name="c", subcore_axis_name="s",
                               num_cores=1)


def _newton_rsqrt(d):
    m = jnp.full_like(d, 1.0)
    em = jnp.full_like(d, 1.0)
    for j in range(1, 23):
        crossed = d >= jnp.float32(float(2 ** j))
        m = m * jnp.where(crossed, jnp.float32(0.7071067811865476),
                          jnp.float32(1.0))
        em = em * jnp.where(crossed, jnp.float32(0.5), jnp.float32(1.0))
    dn = d * em
    y = m * (1.4274 - 0.3015 * dn)
    for _ in range(3):
        y = y * (1.5 - 0.5 * d * y * y)
    return y


@functools.partial(
    pl.kernel,
    mesh=_mesh,
    compiler_params=pltpu.CompilerParams(use_tc_tiling_on_sc=False),
    out_type=[jax.ShapeDtypeStruct((NP, F), jnp.float32)] * 4,
    scratch_types=[
        pltpu.VMEM((2 * CS, BATCH), jnp.int32),
        pltpu.VMEM((2 * CS, BATCH), jnp.int32),
        pltpu.VMEM((2 * CS, BATCH), jnp.int32),
        pltpu.VMEM((2 * CS * BATCH, F), jnp.float32),
        pltpu.VMEM((MS, F), jnp.float32),
        pltpu.VMEM((MS, F), jnp.float32),
        pltpu.VMEM((MS, F), jnp.float32),
        pltpu.VMEM((MS // 8, 8 * F), jnp.float32),
        pltpu.VMEM((BATCH, F), jnp.float32),
        pltpu.VMEM((1, F), jnp.float32),
        pltpu.VMEM_SHARED((ACCR, F), jnp.float32),
        pltpu.SemaphoreType.DMA,
        pltpu.SemaphoreType.DMA,
        pltpu.SemaphoreType.DMA,
        pltpu.SemaphoreType.DMA,
    ],
)
def _sc_mega(src_hbm, dst_hbm, h1_hbm, ones_hbm, b1_hbm,
             disb_hbm, t1_hbm, r_hbm, a2_hbm,
             sidx2, draw2, didx2, rows, bufa, bufb, bufc, bufh, onev, b1v,
             acc, sem, isem, ssem0, ssem1):
    s = lax.axis_index("s")

    def remap(d16, base):
        inr = jnp.logical_and(d16 >= base, d16 < base + NH)
        return jnp.where(inr, d16 - base, NH + (d16 & (DUMP - 1)))

    def load_dst(chunk, h):
        jo = h * CS
        return [pltpu.async_copy(
            dst_hbm.at[pl.ds((chunk * CS + j) * BATCH, BATCH)],
            draw2.at[jo + j], isem) for j in range(CS)]

    def remap_dst(base, h):
        jo = h * CS

        def rbody(g, carry):
            j = jo + (g >> 3)
            u = g & 7
            d16 = remap(draw2[j, pl.ds(u * 16, 16)], base)
            didx2[j, pl.ds(u * 16, 16)] = d16
            return carry

        lax.fori_loop(0, CS * 8, rbody, 0)

    def drain_scatters(h):
        ssem = ssem0 if h == 0 else ssem1
        ro = h * CS * BATCH
        for j in range(CS):
            pltpu.make_async_copy(
                t1_hbm.at[pl.ds(0, BATCH)],
                rows.at[pl.ds(ro + j * BATCH, BATCH)], ssem).wait()

    def deg_half(base, chunk, h):
        ssem = ssem0 if h == 0 else ssem1
        jo = h * CS
        for cp in load_dst(chunk, h):
            cp.wait()
        remap_dst(base, h)
        for j in range(CS):
            pltpu.async_copy(onev, acc.at[didx2.at[jo + j]], ssem, add=True)

    def agg_half(tab_hbm, base, chunk, h):
        ssem = ssem0 if h == 0 else ssem1
        jo = h * CS
        ro = h * CS * BATCH
        icopies = load_dst(chunk, h) + [pltpu.async_copy(
            src_hbm.at[pl.ds((chunk * CS + j) * BATCH, BATCH)],
            sidx2.at[jo + j], isem) for j in range(CS)]
        for cp in icopies:
            cp.wait()
        copies = [
            pltpu.async_copy(tab_hbm.at[sidx2.at[jo + j]],
                             rows.at[pl.ds(ro + j * BATCH, BATCH)], sem)
            for j in range(CS)
        ]
        remap_dst(base, h)
        for cp in copies:
            cp.wait()
        for j in range(CS):
            pltpu.async_copy(rows.at[pl.ds(ro + j * BATCH, BATCH)],
                             acc.at[didx2.at[jo + j]], ssem, add=True)

    def edge_pass(half_fn):
        c0 = s * PER_T
        half_fn(c0, 0)
        half_fn(c0 + 1, 1)

        def body(i, carry):
            drain_scatters(0)
            half_fn(c0 + 2 * i, 0)
            drain_scatters(1)
            half_fn(c0 + 2 * i + 1, 1)
            return carry

        lax.fori_loop(1, PER_T // 2, body, 0)
        drain_scatters(0)
        drain_scatters(1)

        @pl.when(s < EXTRA)
        def _():
            half_fn(NS * PER_T + s, 0)
            drain_scatters(0)

    pltpu.sync_copy(ones_hbm, onev)
    pltpu.sync_copy(b1_hbm, b1v)

    def zero_phase():
        def zb(i, carry):
            bufa[i, :] = jnp.zeros((F,), jnp.float32)
            return carry
        lax.fori_loop(0, MS, zb, 0)
        for piece in range(4):
            pltpu.sync_copy(bufa.at[pl.ds(0, MS)],
                            acc.at[pl.ds(s * 3144 + piece * MS, MS)])
        pltpu.sync_copy(bufa.at[pl.ds(0, 8)],
                        acc.at[pl.ds(s * 3144 + 4 * MS, 8)])

    for p in range(2):
        base = p * NH
        zero_phase()
        plsc.subcore_barrier()
        edge_pass(functools.partial(deg_half, base))
        plsc.subcore_barrier()
        for q in range(4):
            loc = s * MSPAN + q * MS
            glob = base + loc
            pltpu.sync_copy(acc.at[pl.ds(loc, MS)], bufa)
            pltpu.sync_copy(h1_hbm.at[pl.ds(glob // 8, MS // 8)], bufh)

            def cbody(i, carry):
                dis = _newton_rsqrt(bufa[i, :] + 1.0)
                bufa[i, :] = dis
                hv = bufh[i >> 3, pl.ds((i & 7) * F, F)]
                bufc[i, :] = hv * dis
                return carry

            lax.fori_loop(0, MS, cbody, 0)
            pltpu.sync_copy(bufa, disb_hbm.at[pl.ds(glob, MS)])
            pltpu.sync_copy(bufc, t1_hbm.at[pl.ds(glob, MS)])
        plsc.subcore_barrier()

    for p in range(2):
        base = p * NH
        zero_phase()
        plsc.subcore_barrier()
        edge_pass(functools.partial(agg_half, t1_hbm, base))
        plsc.subcore_barrier()
        for q in range(4):
            loc = s * MSPAN + q * MS
            glob = base + loc
            pltpu.sync_copy(acc.at[pl.ds(loc, MS)], bufa)
            pltpu.sync_copy(t1_hbm.at[pl.ds(glob, MS)], bufb)
            pltpu.sync_copy(disb_hbm.at[pl.ds(glob, MS)], bufc)

            def ebody(i, carry):
                dis = bufc[i, :]
                r = dis * jnp.maximum(
                    dis * (bufa[i, :] + bufb[i, :]) + b1v[0, :], 0.0)
                bufa[i, :] = r
                return carry

            lax.fori_loop(0, MS, ebody, 0)
            pltpu.sync_copy(bufa, r_hbm.at[pl.ds(glob, MS)])
        plsc.subcore_barrier()

    for p in range(2):
        base = p * NH
        zero_phase()
        plsc.subcore_barrier()
        edge_pass(functools.partial(agg_half, r_hbm, base))
        plsc.subcore_barrier()
        for q in range(4):
            loc = s * MSPAN + q * MS
            glob = base + loc
            pltpu.sync_copy(acc.at[pl.ds(loc, MS)], bufa)
            pltpu.sync_copy(bufa, a2_hbm.at[pl.ds(glob, MS)])
        plsc.subcore_barrier()


BR = 2048


def _tc1_body(x8_ref, w1b_ref, h_ref):
    h_ref[...] = jnp.dot(x8_ref[...], w1b_ref[...],
                         preferred_element_type=jnp.float32)


def _tc2_body(disb_ref, r_ref, a2_ref, w2b_ref, b2t_ref, out_ref):
    ssum = a2_ref[...] + r_ref[...]
    out_ref[...] = disb_ref[...] * jnp.dot(
        ssum, w2b_ref[...], preferred_element_type=jnp.float32) \
        + b2t_ref[...]


def kernel(x, edge_index, W1, b1, W2, b2):
    src1 = edge_index[0]
    dst1 = edge_index[1]
    ones16 = jnp.ones((BATCH, F), jnp.float32)
    b1r = b1.reshape(1, F)

    nblk = NP // BR
    x8 = x.reshape(N // 8, 8 * D)
    w1big = jnp.kron(jnp.eye(8, dtype=jnp.float32), W1)
    h1p = pl.pallas_call(
        _tc1_body,
        grid=(nblk,),
        in_specs=[
            pl.BlockSpec((BR // 8, 8 * D), lambda i: (i, 0)),
            pl.BlockSpec((8 * D, 8 * F), lambda i: (0, 0)),
        ],
        out_specs=pl.BlockSpec((BR // 8, 8 * F), lambda i: (i, 0)),
        out_shape=jax.ShapeDtypeStruct((NP // 8, 8 * F), jnp.float32),
    )(x8, w1big)

    disb, t1, r, a2 = _sc_mega(src1, dst1, h1p, ones16, b1r)

    w2big = jnp.kron(jnp.eye(8, dtype=jnp.float32), W2)
    b2t = jnp.tile(b2, 8).reshape(1, 8 * F)
    disp = disb.reshape(NP // 8, 8 * F)
    rp = r.reshape(NP // 8, 8 * F)
    a2p = a2.reshape(NP // 8, 8 * F)
    out = pl.pallas_call(
        _tc2_body,
        grid=(nblk,),
        in_specs=[
            pl.BlockSpec((BR // 8, 8 * F), lambda i: (i, 0)),
            pl.BlockSpec((BR // 8, 8 * F), lambda i: (i, 0)),
            pl.BlockSpec((BR // 8, 8 * F), lambda i: (i, 0)),
            pl.BlockSpec((8 * F, 8 * F), lambda i: (0, 0)),
            pl.BlockSpec((1, 8 * F), lambda i: (0, 0)),
        ],
        out_specs=pl.BlockSpec((BR // 8, 8 * F), lambda i: (i, 0)),
        out_shape=jax.ShapeDtypeStruct((NP // 8, 8 * F), jnp.float32),
    )(disp, rp, a2p, w2big, b2t)

    return out.reshape(NP, F)[:N]


# speedup vs baseline: 20.6151x; 1.0006x over previous
import functools

import jax
import jax.numpy as jnp
from jax import lax
from jax.experimental import pallas as pl
from jax.experimental.pallas import tpu as pltpu
from jax.experimental.pallas import tpu_sc as plsc

N = 100000
NP = 100352
E = 3200000
D = 128
F = 16
NH = NP // 2
DUMP = 128
ACCR = NH + DUMP
NS = 16
BATCH = 128
NB = E // BATCH
CS = 4
NCHE = NB // CS
PER_T = NCHE // NS
EXTRA = NCHE - PER_T * NS
MSPAN = NH // NS
MS = MSPAN // 4
ZR = 1048

_mesh = plsc.VectorSubcoreMesh(core_axis_name="c", subcore_axis_name="s",
                               num_cores=1)


def _newton_rsqrt(d):
    m = jnp.full_like(d, 1.0)
    em = jnp.full_like(d, 1.0)
    for j in range(1, 23):
        crossed = d >= jnp.float32(float(2 ** j))
        m = m * jnp.where(crossed, jnp.float32(0.7071067811865476),
                          jnp.float32(1.0))
        em = em * jnp.where(crossed, jnp.float32(0.5), jnp.float32(1.0))
    dn = d * em
    y = m * (1.4274 - 0.3015 * dn)
    for _ in range(3):
        y = y * (1.5 - 0.5 * d * y * y)
    return y


@functools.partial(
    pl.kernel,
    mesh=_mesh,
    compiler_params=pltpu.CompilerParams(use_tc_tiling_on_sc=False),
    out_type=[jax.ShapeDtypeStruct((NP, F), jnp.float32)] * 4,
    scratch_types=[
        pltpu.VMEM((3 * CS, BATCH), jnp.int32),
        pltpu.VMEM((3 * CS, BATCH), jnp.int32),
        pltpu.VMEM((3 * CS, BATCH), jnp.int32),
        pltpu.VMEM((3 * CS * BATCH, F), jnp.float32),
        pltpu.VMEM((MS, F), jnp.float32),
        pltpu.VMEM((MS, F), jnp.float32),
        pltpu.VMEM((MS // 8, 8 * F), jnp.float32),
        pltpu.VMEM((BATCH, F), jnp.float32),
        pltpu.VMEM((1, F), jnp.float32),
        pltpu.VMEM_SHARED((ACCR, F), jnp.float32),
        pltpu.SemaphoreType.DMA,
        pltpu.SemaphoreType.DMA,
        pltpu.SemaphoreType.DMA,
        pltpu.SemaphoreType.DMA,
        pltpu.SemaphoreType.DMA,
    ],
)
def _sc_mega(src_hbm, dst_hbm, h1_hbm, ones_hbm, b1_hbm,
             disb_hbm, t1_hbm, r_hbm, a2_hbm,
             sidx2, draw2, didx2, rows, bufa, bufb, bufh, onev, b1v,
             acc, sem, isem, ssem0, ssem1, ssem2):
    s = lax.axis_index("s")

    def remap(d16, base):
        inr = jnp.logical_and(d16 >= base, d16 < base + NH)
        return jnp.where(inr, d16 - base, NH + (d16 & (DUMP - 1)))

    def load_dst(chunk, h):
        jo = h * CS
        return [pltpu.async_copy(
            dst_hbm.at[pl.ds((chunk * CS + j) * BATCH, BATCH)],
            draw2.at[jo + j], isem) for j in range(CS)]

    def remap_dst(base, h):
        jo = h * CS

        def rbody(g, carry):
            j = jo + (g >> 3)
            u = g & 7
            d16 = remap(draw2[j, pl.ds(u * 16, 16)], base)
            didx2[j, pl.ds(u * 16, 16)] = d16
            return carry

        lax.fori_loop(0, CS * 8, rbody, 0)

    def drain_scatters(h):
        ssem = (ssem0, ssem1, ssem2)[h]
        ro = h * CS * BATCH
        for j in range(CS):
            pltpu.make_async_copy(
                t1_hbm.at[pl.ds(0, BATCH)],
                rows.at[pl.ds(ro + j * BATCH, BATCH)], ssem).wait()

    def deg_half(base, chunk, h):
        ssem = (ssem0, ssem1, ssem2)[h]
        jo = h * CS
        for cp in load_dst(chunk, h):
            cp.wait()
        remap_dst(base, h)
        for j in range(CS):
            pltpu.async_copy(onev, acc.at[didx2.at[jo + j]], ssem, add=True)

    def agg_half(tab_hbm, base, chunk, h):
        ssem = (ssem0, ssem1, ssem2)[h]
        jo = h * CS
        ro = h * CS * BATCH
        icopies = load_dst(chunk, h) + [pltpu.async_copy(
            src_hbm.at[pl.ds((chunk * CS + j) * BATCH, BATCH)],
            sidx2.at[jo + j], isem) for j in range(CS)]
        for cp in icopies:
            cp.wait()
        copies = [
            pltpu.async_copy(tab_hbm.at[sidx2.at[jo + j]],
                             rows.at[pl.ds(ro + j * BATCH, BATCH)], sem)
            for j in range(CS)
        ]
        remap_dst(base, h)
        for cp in copies:
            cp.wait()
        for j in range(CS):
            pltpu.async_copy(rows.at[pl.ds(ro + j * BATCH, BATCH)],
                             acc.at[didx2.at[jo + j]], ssem, add=True)

    def edge_pass(half_fn):
        c0 = s * PER_T
        half_fn(c0, 0)
        half_fn(c0 + 1, 1)
        half_fn(c0 + 2, 2)

        def body(i, carry):
            for h in range(3):
                drain_scatters(h)
                half_fn(c0 + 3 * i + h, h)
            return carry

        lax.fori_loop(1, PER_T // 3, body, 0)
        for h in range(3):
            drain_scatters(h)

        @pl.when(s < EXTRA)
        def _():
            half_fn(NS * PER_T + s, 0)
            drain_scatters(0)

    pltpu.sync_copy(ones_hbm, onev)
    pltpu.sync_copy(b1_hbm, b1v)

    def zero_phase():
        def zb(i, carry):
            bufa[i, :] = jnp.zeros((F,), jnp.float32)
            return carry
        lax.fori_loop(0, MS, zb, 0)
        for piece in range(4):
            pltpu.sync_copy(bufa.at[pl.ds(0, MS)],
                            acc.at[pl.ds(s * 3144 + piece * MS, MS)])
        pltpu.sync_copy(bufa.at[pl.ds(0, 8)],
                        acc.at[pl.ds(s * 3144 + 4 * MS, 8)])

    for p in range(2):
        base = p * NH
        zero_phase()
        plsc.subcore_barrier()
        edge_pass(functools.partial(deg_half, base))
        plsc.subcore_barrier()
        for q in range(4):
            loc = s * MSPAN + q * MS
            glob = base + loc
            pltpu.sync_copy(acc.at[pl.ds(loc, MS)], bufa)
            pltpu.sync_copy(h1_hbm.at[pl.ds(glob // 8, MS // 8)], bufh)

            def cbody(i, carry):
                dis = _newton_rsqrt(bufa[i, :] + 1.0)
                bufa[i, :] = dis
                hv = bufh[i >> 3, pl.ds((i & 7) * F, F)]
                rows[i, :] = hv * dis
                return carry

            lax.fori_loop(0, MS, cbody, 0)
            pltpu.sync_copy(bufa, disb_hbm.at[pl.ds(glob, MS)])
            pltpu.sync_copy(rows.at[pl.ds(0, MS)], t1_hbm.at[pl.ds(glob, MS)])
        plsc.subcore_barrier()

    for p in range(2):
        base = p * NH
        zero_phase()
        plsc.subcore_barrier()
        edge_pass(functools.partial(agg_half, t1_hbm, base))
        plsc.subcore_barrier()
        for q in range(4):
            loc = s * MSPAN + q * MS
            glob = base + loc
            pltpu.sync_copy(acc.at[pl.ds(loc, MS)], bufa)
            pltpu.sync_copy(t1_hbm.at[pl.ds(glob, MS)], bufb)
            pltpu.sync_copy(disb_hbm.at[pl.ds(glob, MS)],
                            rows.at[pl.ds(0, MS)])

            def ebody(i, carry):
                dis = rows[i, :]
                r = dis * jnp.maximum(
                    dis * (bufa[i, :] + bufb[i, :]) + b1v[0, :], 0.0)
                bufa[i, :] = r
                return carry

            lax.fori_loop(0, MS, ebody, 0)
            pltpu.sync_copy(bufa, r_hbm.at[pl.ds(glob, MS)])
        plsc.subcore_barrier()

    for p in range(2):
        base = p * NH
        zero_phase()
        plsc.subcore_barrier()
        edge_pass(functools.partial(agg_half, r_hbm, base))
        plsc.subcore_barrier()
        for q in range(4):
            loc = s * MSPAN + q * MS
            glob = base + loc
            pltpu.sync_copy(acc.at[pl.ds(loc, MS)], bufa)
            pltpu.sync_copy(bufa, a2_hbm.at[pl.ds(glob, MS)])
        plsc.subcore_barrier()


BR = 2048


def _tc1_body(x8_ref, w1b_ref, h_ref):
    h_ref[...] = jnp.dot(x8_ref[...], w1b_ref[...],
                         preferred_element_type=jnp.float32)


def _tc2_body(disb_ref, r_ref, a2_ref, w2b_ref, b2t_ref, out_ref):
    ssum = a2_ref[...] + r_ref[...]
    out_ref[...] = disb_ref[...] * jnp.dot(
        ssum, w2b_ref[...], preferred_element_type=jnp.float32) \
        + b2t_ref[...]


def kernel(x, edge_index, W1, b1, W2, b2):
    src1 = edge_index[0]
    dst1 = edge_index[1]
    ones16 = jnp.ones((BATCH, F), jnp.float32)
    b1r = b1.reshape(1, F)

    nblk = NP // BR
    x8 = x.reshape(N // 8, 8 * D)
    w1big = jnp.kron(jnp.eye(8, dtype=jnp.float32), W1)
    h1p = pl.pallas_call(
        _tc1_body,
        grid=(nblk,),
        in_specs=[
            pl.BlockSpec((BR // 8, 8 * D), lambda i: (i, 0)),
            pl.BlockSpec((8 * D, 8 * F), lambda i: (0, 0)),
        ],
        out_specs=pl.BlockSpec((BR // 8, 8 * F), lambda i: (i, 0)),
        out_shape=jax.ShapeDtypeStruct((NP // 8, 8 * F), jnp.float32),
    )(x8, w1big)

    disb, t1, r, a2 = _sc_mega(src1, dst1, h1p, ones16, b1r)

    w2big = jnp.kron(jnp.eye(8, dtype=jnp.float32), W2)
    b2t = jnp.tile(b2, 8).reshape(1, 8 * F)
    disp = disb.reshape(NP // 8, 8 * F)
    rp = r.reshape(NP // 8, 8 * F)
    a2p = a2.reshape(NP // 8, 8 * F)
    out = pl.pallas_call(
        _tc2_body,
        grid=(nblk,),
        in_specs=[
            pl.BlockSpec((BR // 8, 8 * F), lambda i: (i, 0)),
            pl.BlockSpec((BR // 8, 8 * F), lambda i: (i, 0)),
            pl.BlockSpec((BR // 8, 8 * F), lambda i: (i, 0)),
            pl.BlockSpec((8 * F, 8 * F), lambda i: (0, 0)),
            pl.BlockSpec((1, 8 * F), lambda i: (0, 0)),
        ],
        out_specs=pl.BlockSpec((BR // 8, 8 * F), lambda i: (i, 0)),
        out_shape=jax.ShapeDtypeStruct((NP // 8, 8 * F), jnp.float32),
    )(disp, rp, a2p, w2big, b2t)

    return out.reshape(NP, F)[:N]


# speedup vs baseline: 25.8920x; 1.2560x over previous
import functools

import jax
import jax.numpy as jnp
from jax import lax
from jax.experimental import pallas as pl
from jax.experimental.pallas import tpu as pltpu
from jax.experimental.pallas import tpu_sc as plsc

N = 100000
NP = 100352
E = 3200000
D = 128
F = 16
NH = NP // 2
DUMP = 128
ACCR = NH + DUMP
NS = 16
BATCH = 128
NB = E // BATCH
CS = 4
NCHE = NB // CS
PER_T = NCHE // NS
EXTRA = NCHE - PER_T * NS
MSPAN = NH // NS
MS = MSPAN // 4
ZR = 1048

_mesh = plsc.VectorSubcoreMesh(core_axis_name="c", subcore_axis_name="s",
                               num_cores=1)


def _newton_rsqrt(d):
    m = jnp.full_like(d, 1.0)
    em = jnp.full_like(d, 1.0)
    for j in range(1, 23):
        crossed = d >= jnp.float32(float(2 ** j))
        m = m * jnp.where(crossed, jnp.float32(0.7071067811865476),
                          jnp.float32(1.0))
        em = em * jnp.where(crossed, jnp.float32(0.5), jnp.float32(1.0))
    dn = d * em
    y = m * (1.4274 - 0.3015 * dn)
    for _ in range(3):
        y = y * (1.5 - 0.5 * d * y * y)
    return y


@functools.partial(
    pl.kernel,
    mesh=_mesh,
    compiler_params=pltpu.CompilerParams(use_tc_tiling_on_sc=False),
    out_type=[jax.ShapeDtypeStruct((NP, F), jnp.float32)] * 4,
    scratch_types=[
        pltpu.VMEM((3 * CS, BATCH), jnp.int32),
        pltpu.VMEM((3 * CS, BATCH), jnp.int32),
        pltpu.VMEM((3 * CS, BATCH), jnp.int32),
        pltpu.VMEM((3 * CS * BATCH, F), jnp.float32),
        pltpu.VMEM((MS, F), jnp.float32),
        pltpu.VMEM((MS, F), jnp.float32),
        pltpu.VMEM((MS // 8, 8 * F), jnp.float32),
        pltpu.VMEM((BATCH, F), jnp.float32),
        pltpu.VMEM((1, F), jnp.float32),
        pltpu.VMEM_SHARED((ACCR, F), jnp.float32),
        pltpu.SemaphoreType.DMA,
        pltpu.SemaphoreType.DMA,
        pltpu.SemaphoreType.DMA,
        pltpu.SemaphoreType.DMA,
        pltpu.SemaphoreType.DMA,
        pltpu.SemaphoreType.DMA,
        pltpu.SemaphoreType.DMA,
    ],
)
def _sc_mega(src_hbm, dst_hbm, h1_hbm, ones_hbm, b1_hbm,
             disb_hbm, t1_hbm, r_hbm, a2_hbm,
             sidx2, draw2, didx2, rows, bufa, bufb, bufh, onev, b1v,
             acc, sem, isem0, isem1, isem2, ssem0, ssem1, ssem2):
    s = lax.axis_index("s")

    def remap(d16, base):
        inr = jnp.logical_and(d16 >= base, d16 < base + NH)
        return jnp.where(inr, d16 - base, NH + (d16 & (DUMP - 1)))

    def fire_idx(chunk, h):
        isem = (isem0, isem1, isem2)[h]
        jo = h * CS
        for j in range(CS):
            pltpu.async_copy(
                dst_hbm.at[pl.ds((chunk * CS + j) * BATCH, BATCH)],
                draw2.at[jo + j], isem)
            pltpu.async_copy(
                src_hbm.at[pl.ds((chunk * CS + j) * BATCH, BATCH)],
                sidx2.at[jo + j], isem)

    def wait_idx(h):
        isem = (isem0, isem1, isem2)[h]
        jo = h * CS
        for j in range(CS):
            pltpu.make_async_copy(
                dst_hbm.at[pl.ds(0, BATCH)], draw2.at[jo + j], isem).wait()
            pltpu.make_async_copy(
                src_hbm.at[pl.ds(0, BATCH)], sidx2.at[jo + j], isem).wait()

    def remap_dst(base, h):
        jo = h * CS

        def rbody(g, carry):
            j = jo + (g >> 3)
            u = g & 7
            d16 = remap(draw2[j, pl.ds(u * 16, 16)], base)
            didx2[j, pl.ds(u * 16, 16)] = d16
            return carry

        lax.fori_loop(0, CS * 8, rbody, 0)

    def drain_scatters(h):
        ssem = (ssem0, ssem1, ssem2)[h]
        ro = h * CS * BATCH
        for j in range(CS):
            pltpu.make_async_copy(
                t1_hbm.at[pl.ds(0, BATCH)],
                rows.at[pl.ds(ro + j * BATCH, BATCH)], ssem).wait()

    def deg_half(base, chunk, h, nxt):
        ssem = (ssem0, ssem1, ssem2)[h]
        jo = h * CS
        wait_idx(h)
        remap_dst(base, h)
        fire_idx(nxt, (h + 1) % 3)
        for j in range(CS):
            pltpu.async_copy(onev, acc.at[didx2.at[jo + j]], ssem, add=True)

    def agg_half(tab_hbm, base, chunk, h, nxt):
        ssem = (ssem0, ssem1, ssem2)[h]
        jo = h * CS
        ro = h * CS * BATCH
        wait_idx(h)
        copies = [
            pltpu.async_copy(tab_hbm.at[sidx2.at[jo + j]],
                             rows.at[pl.ds(ro + j * BATCH, BATCH)], sem)
            for j in range(CS)
        ]
        remap_dst(base, h)
        fire_idx(nxt, (h + 1) % 3)
        for cp in copies:
            cp.wait()
        for j in range(CS):
            pltpu.async_copy(rows.at[pl.ds(ro + j * BATCH, BATCH)],
                             acc.at[didx2.at[jo + j]], ssem, add=True)

    def edge_pass(half_fn):
        c0 = s * PER_T
        fire_idx(c0, 0)
        half_fn(c0, 0, c0 + 1)
        half_fn(c0 + 1, 1, c0 + 2)
        half_fn(c0 + 2, 2, c0 + 3)

        def body(i, carry):
            for h in range(3):
                chunk = c0 + 3 * i + h
                drain_scatters(h)
                half_fn(chunk, h, jnp.minimum(chunk + 1, NCHE - 1))
            return carry

        lax.fori_loop(1, PER_T // 3, body, 0)
        wait_idx(0)
        for h in range(3):
            drain_scatters(h)

        @pl.when(s < EXTRA)
        def _():
            fire_idx(NS * PER_T + s, 0)
            half_fn(NS * PER_T + s, 0, NCHE - 1)
            wait_idx(1)
            drain_scatters(0)

    pltpu.sync_copy(ones_hbm, onev)
    pltpu.sync_copy(b1_hbm, b1v)

    def zero_phase():
        def zb(i, carry):
            bufa[i, :] = jnp.zeros((F,), jnp.float32)
            return carry
        lax.fori_loop(0, MS, zb, 0)
        for piece in range(4):
            pltpu.sync_copy(bufa.at[pl.ds(0, MS)],
                            acc.at[pl.ds(s * 3144 + piece * MS, MS)])
        pltpu.sync_copy(bufa.at[pl.ds(0, 8)],
                        acc.at[pl.ds(s * 3144 + 4 * MS, 8)])

    for p in range(2):
        base = p * NH
        zero_phase()
        plsc.subcore_barrier()
        edge_pass(functools.partial(deg_half, base))
        plsc.subcore_barrier()
        for q in range(4):
            loc = s * MSPAN + q * MS
            glob = base + loc
            pltpu.sync_copy(acc.at[pl.ds(loc, MS)], bufa)
            pltpu.sync_copy(h1_hbm.at[pl.ds(glob // 8, MS // 8)], bufh)

            def cbody(i, carry):
                dis = _newton_rsqrt(bufa[i, :] + 1.0)
                bufa[i, :] = dis
                hv = bufh[i >> 3, pl.ds((i & 7) * F, F)]
                rows[i, :] = hv * dis
                return carry

            lax.fori_loop(0, MS, cbody, 0)
            pltpu.sync_copy(bufa, disb_hbm.at[pl.ds(glob, MS)])
            pltpu.sync_copy(rows.at[pl.ds(0, MS)], t1_hbm.at[pl.ds(glob, MS)])
        plsc.subcore_barrier()

    for p in range(2):
        base = p * NH
        zero_phase()
        plsc.subcore_barrier()
        edge_pass(functools.partial(agg_half, t1_hbm, base))
        plsc.subcore_barrier()
        for q in range(4):
            loc = s * MSPAN + q * MS
            glob = base + loc
            pltpu.sync_copy(acc.at[pl.ds(loc, MS)], bufa)
            pltpu.sync_copy(t1_hbm.at[pl.ds(glob, MS)], bufb)
            pltpu.sync_copy(disb_hbm.at[pl.ds(glob, MS)],
                            rows.at[pl.ds(0, MS)])

            def ebody(i, carry):
                dis = rows[i, :]
                r = dis * jnp.maximum(
                    dis * (bufa[i, :] + bufb[i, :]) + b1v[0, :], 0.0)
                bufa[i, :] = r
                return carry

            lax.fori_loop(0, MS, ebody, 0)
            pltpu.sync_copy(bufa, r_hbm.at[pl.ds(glob, MS)])
        plsc.subcore_barrier()

    for p in range(2):
        base = p * NH
        zero_phase()
        plsc.subcore_barrier()
        edge_pass(functools.partial(agg_half, r_hbm, base))
        plsc.subcore_barrier()
        for q in range(4):
            loc = s * MSPAN + q * MS
            glob = base + loc
            pltpu.sync_copy(acc.at[pl.ds(loc, MS)], bufa)
            pltpu.sync_copy(bufa, a2_hbm.at[pl.ds(glob, MS)])
        plsc.subcore_barrier()


BR = 2048


def _tc1_body(x8_ref, w1b_ref, h_ref):
    h_ref[...] = jnp.dot(x8_ref[...], w1b_ref[...],
                         preferred_element_type=jnp.float32)


def _tc2_body(disb_ref, r_ref, a2_ref, w2b_ref, b2t_ref, out_ref):
    ssum = a2_ref[...] + r_ref[...]
    out_ref[...] = disb_ref[...] * jnp.dot(
        ssum, w2b_ref[...], preferred_element_type=jnp.float32) \
        + b2t_ref[...]


def kernel(x, edge_index, W1, b1, W2, b2):
    src1 = edge_index[0]
    dst1 = edge_index[1]
    ones16 = jnp.ones((BATCH, F), jnp.float32)
    b1r = b1.reshape(1, F)

    nblk = NP // BR
    x8 = x.reshape(N // 8, 8 * D)
    w1big = jnp.kron(jnp.eye(8, dtype=jnp.float32), W1)
    h1p = pl.pallas_call(
        _tc1_body,
        grid=(nblk,),
        in_specs=[
            pl.BlockSpec((BR // 8, 8 * D), lambda i: (i, 0)),
            pl.BlockSpec((8 * D, 8 * F), lambda i: (0, 0)),
        ],
        out_specs=pl.BlockSpec((BR // 8, 8 * F), lambda i: (i, 0)),
        out_shape=jax.ShapeDtypeStruct((NP // 8, 8 * F), jnp.float32),
    )(x8, w1big)

    disb, t1, r, a2 = _sc_mega(src1, dst1, h1p, ones16, b1r)

    w2big = jnp.kron(jnp.eye(8, dtype=jnp.float32), W2)
    b2t = jnp.tile(b2, 8).reshape(1, 8 * F)
    disp = disb.reshape(NP // 8, 8 * F)
    rp = r.reshape(NP // 8, 8 * F)
    a2p = a2.reshape(NP // 8, 8 * F)
    out = pl.pallas_call(
        _tc2_body,
        grid=(nblk,),
        in_specs=[
            pl.BlockSpec((BR // 8, 8 * F), lambda i: (i, 0)),
            pl.BlockSpec((BR // 8, 8 * F), lambda i: (i, 0)),
            pl.BlockSpec((BR // 8, 8 * F), lambda i: (i, 0)),
            pl.BlockSpec((8 * F, 8 * F), lambda i: (0, 0)),
            pl.BlockSpec((1, 8 * F), lambda i: (0, 0)),
        ],
        out_specs=pl.BlockSpec((BR // 8, 8 * F), lambda i: (i, 0)),
        out_shape=jax.ShapeDtypeStruct((NP // 8, 8 * F), jnp.float32),
    )(disp, rp, a2p, w2big, b2t)

    return out.reshape(NP, F)[:N]
